# baseline (speedup 1.0000x reference)
import functools
import jax
import jax.numpy as jnp
from jax import lax
from jax.experimental import pallas as pl
from jax.experimental.pallas import tpu as pltpu
from jax.experimental.pallas import tpu_sc as plsc

TOKENS = 8192
INPUT_DIM = 1024
HIDDEN_DIM = 256
POOL_SIZE = 16384
K = 1024
MIN_P = 100.0
MAX_P = 1024.0

ROW_A = 256
ROW_B = 128
ROW_D = 128
MAXI32 = 0x7FFFFFFF


def _hs_complexity_body(x_ref, W1_ref, b1_ref, W2_ref, b2_ref, S1_ref, bs1_ref,
                        hs_ref, comp_ref):
    x = x_ref[...]
    h = jnp.maximum(jnp.dot(x, W1_ref[...], preferred_element_type=jnp.float32)
                    + b1_ref[...], 0.0)
    logit = jnp.dot(h, W2_ref[...], preferred_element_type=jnp.float32) + b2_ref[...]
    comp_ref[...] = jax.nn.sigmoid(logit)
    hs_ref[...] = jnp.maximum(
        jnp.dot(x, S1_ref[...], preferred_element_type=jnp.float32) + bs1_ref[...], 0.0)


def _f32_to_key(s):
    b = jax.lax.bitcast_convert_type(s, jnp.int32)
    return jnp.where(b >= 0, b, MAXI32 - b)


def _key_to_f32(k):
    b = jnp.where(k >= 0, k, MAXI32 - k)
    return jax.lax.bitcast_convert_type(b, jnp.float32)


def _keys_bisect_body(hs_ref, S2_ref, bs2_ref, keys_ref, t_ref, quota_ref,
                      lo_ref, hi_ref, fr_ref):
    scores = (jnp.dot(hs_ref[...], S2_ref[...],
                      preferred_element_type=jnp.float32) + bs2_ref[...])
    keys = _f32_to_key(scores)
    keys_ref[...] = keys

    lo_ref[...] = jnp.min(keys, axis=1, keepdims=True)
    hi_ref[...] = jnp.max(keys, axis=1, keepdims=True) + 1
    fr_ref[...] = jnp.zeros(fr_ref.shape, jnp.int32)

    def cond(carry):
        it, done = carry
        return jnp.logical_and(it < 34, done == 0)

    def body(carry):
        it, done = carry
        lo = lo_ref[...]
        hi = hi_ref[...]
        fr = fr_ref[...]
        mid = (lo >> 1) + (hi >> 1) + (lo & hi & 1)
        cnt = jnp.sum((keys >= mid).astype(jnp.int32), axis=1, keepdims=True)
        active = (fr == 0) & (hi - lo != 1)
        ge = cnt >= K
        new_lo = jnp.where(active & ge, mid, lo)
        new_hi = jnp.where(active & jnp.logical_not(ge), mid, hi)
        new_fr = fr | (active & (cnt == K)).astype(jnp.int32)
        lo_ref[...] = new_lo
        hi_ref[...] = new_hi
        fr_ref[...] = new_fr
        all_done = jnp.all((new_fr != 0) | (new_hi - new_lo == 1))
        return it + 1, all_done.astype(jnp.int32)

    jax.lax.while_loop(cond, body, (jnp.int32(0), jnp.int32(0)))

    lo = lo_ref[...]
    frozen = fr_ref[...] != 0
    sel = keys >= lo
    minsel = jnp.min(jnp.where(sel, keys, MAXI32), axis=1, keepdims=True)
    t = jnp.where(frozen, minsel, lo)
    cnt_gt = jnp.sum((keys > t).astype(jnp.int32), axis=1, keepdims=True)
    quota = K - cnt_gt
    t_ref[...] = jnp.broadcast_to(t, t_ref.shape)
    quota_ref[...] = jnp.broadcast_to(quota, quota_ref.shape)


NW = 32
RPW = TOKENS // NW
OPAD = K + 16


GROUP = 8


def _compact_body(keys_hbm, t_hbm, q_hbm, ck_hbm, ci_hbm,
                  kbuf0, kbuf1, tbuf, qbuf, outk, outi, insem):
    c = lax.axis_index("c")
    s = lax.axis_index("s")
    w = s * 2 + c
    row0 = w * RPW
    iota = lax.iota(jnp.int32, 16)
    pltpu.sync_copy(t_hbm.at[pl.ds(row0 * 16, RPW * 16)], tbuf)
    pltpu.sync_copy(q_hbm.at[pl.ds(row0 * 16, RPW * 16)], qbuf)
    kbufs = (kbuf0, kbuf1)
    pltpu.make_async_copy(keys_hbm.at[row0], kbuf0, insem.at[0]).start()

    zero16 = jnp.zeros((16,), jnp.int32)

    def gbody(g, _unused):
        for rr in range(GROUP):
            m = g * GROUP + rr
            p = rr % 2
            np_ = (rr + 1) % 2
            row = row0 + m
            nxt = row0 + lax.rem(m + 1, RPW)
            pltpu.make_async_copy(keys_hbm.at[nxt], kbufs[np_],
                                  insem.at[np_]).start()
            pltpu.make_async_copy(keys_hbm.at[row], kbufs[p],
                                  insem.at[p]).wait()
            t = tbuf[pl.ds(m * 16, 16)]
            q = qbuf[pl.ds(m * 16, 16)]
            krow = kbufs[p]

            def vbody(j, carry):
                acc, eqb = carry
                v = krow[pl.ds(j * 16, 16)]
                m_gt = v > t
                m_eq = v == t
                rank = plsc.cumsum(m_eq.astype(jnp.int32)) + eqb
                m_sel = m_gt | (m_eq & (rank <= q))
                sci = m_sel.astype(jnp.int32)
                pos = acc + plsc.cumsum(sci) - sci + (rr * K)
                plsc.store_scatter(outk, [pos], v, mask=m_sel)
                plsc.store_scatter(outi, [pos], iota + j * 16, mask=m_sel)
                return (acc + plsc.all_reduce_population_count(m_sel),
                        eqb + plsc.all_reduce_population_count(m_eq))

            lax.fori_loop(0, POOL_SIZE // 16, vbody, (zero16, zero16),
                          unroll=4)
        base = (row0 + g * GROUP) * K
        pltpu.sync_copy(outk, ck_hbm.at[pl.ds(base, GROUP * K)])
        pltpu.sync_copy(outi, ci_hbm.at[pl.ds(base, GROUP * K)])
        return _unused

    lax.fori_loop(0, RPW // GROUP, gbody, jnp.int32(0))
    pltpu.make_async_copy(keys_hbm.at[row0], kbuf0, insem.at[0]).wait()


def _compact_call(keys, t_b, quota_b):
    mesh = plsc.VectorSubcoreMesh(core_axis_name="c", subcore_axis_name="s", num_cores=2, num_subcores=16)
    f = pl.kernel(
        _compact_body,
        out_type=[
            jax.ShapeDtypeStruct((TOKENS * K,), jnp.int32),
            jax.ShapeDtypeStruct((TOKENS * K,), jnp.int32),
        ],
        mesh=mesh,
        compiler_params=pltpu.CompilerParams(needs_layout_passes=False),
        scratch_types=[
            pltpu.VMEM((POOL_SIZE,), jnp.int32),
            pltpu.VMEM((POOL_SIZE,), jnp.int32),
            pltpu.VMEM((RPW * 16,), jnp.int32),
            pltpu.VMEM((RPW * 16,), jnp.int32),
            pltpu.VMEM((GROUP * K,), jnp.int32),
            pltpu.VMEM((GROUP * K,), jnp.int32),
            pltpu.SemaphoreType.DMA((2,)),
        ],
    )
    ck, ci = f(keys, t_b.reshape(-1), quota_b.reshape(-1))
    return ck.reshape(TOKENS, K), ci.reshape(TOKENS, K)


def _bitonic_sort_desc(keys, idx):
    R, N = keys.shape
    lane = jax.lax.broadcasted_iota(jnp.int32, (R, N), 1)
    k = 2
    while k <= N:
        j = k // 2
        while j >= 1:
            low = (lane & j) == 0
            pk = jnp.where(low, pltpu.roll(keys, N - j, 1), pltpu.roll(keys, j, 1))
            pi = jnp.where(low, pltpu.roll(idx, N - j, 1), pltpu.roll(idx, j, 1))
            self_beats = (keys > pk) | ((keys == pk) & (idx < pi))
            want_max = ((lane & k) == 0) == low
            take_self = want_max == self_beats
            keys = jnp.where(take_self, keys, pk)
            idx = jnp.where(take_self, idx, pi)
            j //= 2
        k *= 2
    return keys, idx


def _sort_finalize_body(ckeys_ref, cidx_ref, comp_ref,
                        out_idx_ref, out_w_ref, out_mask_ref):
    keys, idx = _bitonic_sort_desc(ckeys_ref[...], cidx_ref[...])
    ts = _key_to_f32(keys)
    m = ts[:, 0:1]
    e = jnp.exp(ts - m)
    w = e / jnp.sum(e, axis=1, keepdims=True)
    comp = comp_ref[...]
    budgets = jnp.round(
        jnp.clip(MIN_P + (MAX_P - MIN_P) * comp * comp, MIN_P, MAX_P)
    ).astype(jnp.int32)
    pos = jax.lax.broadcasted_iota(jnp.int32, out_mask_ref.shape, 1)
    mask = (pos < budgets).astype(jnp.float32)
    out_idx_ref[...] = idx
    out_w_ref[...] = w * mask
    out_mask_ref[...] = mask


def kernel(x, W1, b1, W2, b2, S1, bs1, S2, bs2):
    n_ra = TOKENS // ROW_A
    hs, comp = pl.pallas_call(
        _hs_complexity_body,
        grid=(n_ra,),
        in_specs=[
            pl.BlockSpec((ROW_A, INPUT_DIM), lambda i: (i, 0)),
            pl.BlockSpec((INPUT_DIM, 128), lambda i: (0, 0)),
            pl.BlockSpec((128,), lambda i: (0,)),
            pl.BlockSpec((128, 1), lambda i: (0, 0)),
            pl.BlockSpec((1,), lambda i: (0,)),
            pl.BlockSpec((INPUT_DIM, HIDDEN_DIM), lambda i: (0, 0)),
            pl.BlockSpec((HIDDEN_DIM,), lambda i: (0,)),
        ],
        out_specs=[
            pl.BlockSpec((ROW_A, HIDDEN_DIM), lambda i: (i, 0)),
            pl.BlockSpec((ROW_A, 1), lambda i: (i, 0)),
        ],
        out_shape=[
            jax.ShapeDtypeStruct((TOKENS, HIDDEN_DIM), jnp.float32),
            jax.ShapeDtypeStruct((TOKENS, 1), jnp.float32),
        ],
    )(x, W1, b1, W2, b2, S1, bs1)

    n_rb = TOKENS // ROW_B
    keys, t_b, quota_b = pl.pallas_call(
        _keys_bisect_body,
        grid=(n_rb,),
        in_specs=[
            pl.BlockSpec((ROW_B, HIDDEN_DIM), lambda i: (i, 0)),
            pl.BlockSpec((HIDDEN_DIM, POOL_SIZE), lambda i: (0, 0)),
            pl.BlockSpec((POOL_SIZE,), lambda i: (0,)),
        ],
        out_specs=[
            pl.BlockSpec((ROW_B, POOL_SIZE), lambda i: (i, 0)),
            pl.BlockSpec((ROW_B, 16), lambda i: (i, 0)),
            pl.BlockSpec((ROW_B, 16), lambda i: (i, 0)),
        ],
        out_shape=[
            jax.ShapeDtypeStruct((TOKENS, POOL_SIZE), jnp.int32),
            jax.ShapeDtypeStruct((TOKENS, 16), jnp.int32),
            jax.ShapeDtypeStruct((TOKENS, 16), jnp.int32),
        ],
        scratch_shapes=[
            pltpu.VMEM((ROW_B, 1), jnp.int32),
            pltpu.VMEM((ROW_B, 1), jnp.int32),
            pltpu.VMEM((ROW_B, 1), jnp.int32),
        ],
    )(hs, S2, bs2)

    ckeys, cidx = _compact_call(keys, t_b, quota_b)

    n_rd = TOKENS // ROW_D
    out_idx, out_w, out_mask = pl.pallas_call(
        _sort_finalize_body,
        grid=(n_rd,),
        in_specs=[
            pl.BlockSpec((ROW_D, K), lambda i: (i, 0)),
            pl.BlockSpec((ROW_D, K), lambda i: (i, 0)),
            pl.BlockSpec((ROW_D, 1), lambda i: (i, 0)),
        ],
        out_specs=[
            pl.BlockSpec((ROW_D, K), lambda i: (i, 0)),
            pl.BlockSpec((ROW_D, K), lambda i: (i, 0)),
            pl.BlockSpec((ROW_D, K), lambda i: (i, 0)),
        ],
        out_shape=[
            jax.ShapeDtypeStruct((TOKENS, K), jnp.int32),
            jax.ShapeDtypeStruct((TOKENS, K), jnp.float32),
            jax.ShapeDtypeStruct((TOKENS, K), jnp.float32),
        ],
    )(ckeys, cidx, comp)
    return out_idx, out_w, out_mask, comp


# speedup vs baseline: 1.5074x; 1.5074x over previous
import functools
import jax
import jax.numpy as jnp
from jax import lax
from jax.experimental import pallas as pl
from jax.experimental.pallas import tpu as pltpu
from jax.experimental.pallas import tpu_sc as plsc

TOKENS = 8192
INPUT_DIM = 1024
HIDDEN_DIM = 256
POOL_SIZE = 16384
K = 1024
MIN_P = 100.0
MAX_P = 1024.0

ROW_A = 256
ROW_B = 128
ROW_D = 128
MAXI32 = 0x7FFFFFFF


def _hs_complexity_body(x_ref, W1_ref, b1_ref, W2_ref, b2_ref, S1_ref, bs1_ref,
                        hs_ref, comp_ref):
    x = x_ref[...]
    h = jnp.maximum(jnp.dot(x, W1_ref[...], preferred_element_type=jnp.float32)
                    + b1_ref[...], 0.0)
    logit = jnp.dot(h, W2_ref[...], preferred_element_type=jnp.float32) + b2_ref[...]
    comp_ref[...] = jax.nn.sigmoid(logit)
    hs_ref[...] = jnp.maximum(
        jnp.dot(x, S1_ref[...], preferred_element_type=jnp.float32) + bs1_ref[...], 0.0)


def _f32_to_key(s):
    b = jax.lax.bitcast_convert_type(s, jnp.int32)
    return jnp.where(b >= 0, b, MAXI32 - b)


def _key_to_f32(k):
    b = jnp.where(k >= 0, k, MAXI32 - k)
    return jax.lax.bitcast_convert_type(b, jnp.float32)


def _keys_bisect_body(hs_ref, S2_ref, bs2_ref, keys_ref, t_ref, quota_ref,
                      lo_ref, hi_ref, fr_ref):
    scores = (jnp.dot(hs_ref[...], S2_ref[...],
                      preferred_element_type=jnp.float32) + bs2_ref[...])
    keys = _f32_to_key(scores)
    keys_ref[...] = keys

    lo_ref[...] = jnp.min(keys, axis=1, keepdims=True)
    hi_ref[...] = jnp.max(keys, axis=1, keepdims=True) + 1
    fr_ref[...] = jnp.zeros(fr_ref.shape, jnp.int32)

    def cond(carry):
        it, done = carry
        return jnp.logical_and(it < 34, done == 0)

    def body(carry):
        it, done = carry
        lo = lo_ref[...]
        hi = hi_ref[...]
        fr = fr_ref[...]
        mid = (lo >> 1) + (hi >> 1) + (lo & hi & 1)
        cnt = jnp.sum((keys >= mid).astype(jnp.int32), axis=1, keepdims=True)
        active = (fr == 0) & (hi - lo != 1)
        ge = cnt >= K
        new_lo = jnp.where(active & ge, mid, lo)
        new_hi = jnp.where(active & jnp.logical_not(ge), mid, hi)
        new_fr = fr | (active & (cnt == K)).astype(jnp.int32)
        lo_ref[...] = new_lo
        hi_ref[...] = new_hi
        fr_ref[...] = new_fr
        all_done = jnp.all((new_fr != 0) | (new_hi - new_lo == 1))
        return it + 1, all_done.astype(jnp.int32)

    jax.lax.while_loop(cond, body, (jnp.int32(0), jnp.int32(0)))

    lo = lo_ref[...]
    frozen = fr_ref[...] != 0
    sel = keys >= lo
    minsel = jnp.min(jnp.where(sel, keys, MAXI32), axis=1, keepdims=True)
    t = jnp.where(frozen, minsel, lo)
    cnt_gt = jnp.sum((keys > t).astype(jnp.int32), axis=1, keepdims=True)
    quota = K - cnt_gt
    t_ref[...] = jnp.broadcast_to(t, t_ref.shape)
    quota_ref[...] = jnp.broadcast_to(quota, quota_ref.shape)


NW = 32
RPW = TOKENS // NW
OPAD = K + 16


GROUP = 8


def _compact_body(keys_hbm, t_hbm, q_hbm, ck_hbm, ci_hbm,
                  kbuf0, kbuf1, tbuf, qbuf, outk, outi, insem):
    c = lax.axis_index("c")
    s = lax.axis_index("s")
    w = s * 2 + c
    row0 = w * RPW
    iota = lax.iota(jnp.int32, 16)
    pltpu.sync_copy(t_hbm.at[pl.ds(row0 * 16, RPW * 16)], tbuf)
    pltpu.sync_copy(q_hbm.at[pl.ds(row0 * 16, RPW * 16)], qbuf)
    kbufs = (kbuf0, kbuf1)
    pltpu.make_async_copy(keys_hbm.at[row0], kbuf0, insem.at[0]).start()

    zero16 = jnp.zeros((16,), jnp.int32)

    def gbody(g, _unused):
        for rr in range(GROUP):
            m = g * GROUP + rr
            p = rr % 2
            np_ = (rr + 1) % 2
            row = row0 + m
            nxt = row0 + lax.rem(m + 1, RPW)
            pltpu.make_async_copy(keys_hbm.at[nxt], kbufs[np_],
                                  insem.at[np_]).start()
            pltpu.make_async_copy(keys_hbm.at[row], kbufs[p],
                                  insem.at[p]).wait()
            t = tbuf[pl.ds(m * 16, 16)]
            q = qbuf[pl.ds(m * 16, 16)]
            krow = kbufs[p]

            def vbody(j, carry):
                acc, eqb = carry
                v = krow[pl.ds(j * 16, 16)]
                m_gt = v > t
                m_eq = v == t
                c2 = plsc.cumsum(m_gt.astype(jnp.int32) * 65536
                                 + m_eq.astype(jnp.int32))
                cgt = c2 >> 16
                ceq = c2 & 0xFFFF
                rank = ceq + eqb
                m_sel = m_gt | (m_eq & (rank <= q))
                keep_incl = jnp.minimum(rank, q) - jnp.minimum(eqb, q)
                pos = (acc + cgt + keep_incl - m_sel.astype(jnp.int32)
                       + (rr * K))
                plsc.store_scatter(outk, [pos], v, mask=m_sel)
                plsc.store_scatter(outi, [pos], iota + j * 16, mask=m_sel)
                return (acc + plsc.all_reduce_population_count(m_sel),
                        eqb + plsc.all_reduce_population_count(m_eq))

            plsc.parallel_loop(0, POOL_SIZE // 16, 1, unroll=4,
                               carry=(zero16, zero16))(vbody)
        base = (row0 + g * GROUP) * K
        pltpu.sync_copy(outk, ck_hbm.at[pl.ds(base, GROUP * K)])
        pltpu.sync_copy(outi, ci_hbm.at[pl.ds(base, GROUP * K)])
        return _unused

    lax.fori_loop(0, RPW // GROUP, gbody, jnp.int32(0))
    pltpu.make_async_copy(keys_hbm.at[row0], kbuf0, insem.at[0]).wait()


def _compact_call(keys, t_b, quota_b):
    mesh = plsc.VectorSubcoreMesh(core_axis_name="c", subcore_axis_name="s", num_cores=2, num_subcores=16)
    f = pl.kernel(
        _compact_body,
        out_type=[
            jax.ShapeDtypeStruct((TOKENS * K,), jnp.int32),
            jax.ShapeDtypeStruct((TOKENS * K,), jnp.int32),
        ],
        mesh=mesh,
        compiler_params=pltpu.CompilerParams(needs_layout_passes=False),
        scratch_types=[
            pltpu.VMEM((POOL_SIZE,), jnp.int32),
            pltpu.VMEM((POOL_SIZE,), jnp.int32),
            pltpu.VMEM((RPW * 16,), jnp.int32),
            pltpu.VMEM((RPW * 16,), jnp.int32),
            pltpu.VMEM((GROUP * K,), jnp.int32),
            pltpu.VMEM((GROUP * K,), jnp.int32),
            pltpu.SemaphoreType.DMA((2,)),
        ],
    )
    ck, ci = f(keys, t_b.reshape(-1), quota_b.reshape(-1))
    return ck.reshape(TOKENS, K), ci.reshape(TOKENS, K)


def _bitonic_sort_desc(keys, idx):
    R, N = keys.shape
    lane = jax.lax.broadcasted_iota(jnp.int32, (R, N), 1)
    k = 2
    while k <= N:
        j = k // 2
        while j >= 1:
            low = (lane & j) == 0
            pk = jnp.where(low, pltpu.roll(keys, N - j, 1), pltpu.roll(keys, j, 1))
            pi = jnp.where(low, pltpu.roll(idx, N - j, 1), pltpu.roll(idx, j, 1))
            self_beats = (keys > pk) | ((keys == pk) & (idx < pi))
            want_max = ((lane & k) == 0) == low
            take_self = want_max == self_beats
            keys = jnp.where(take_self, keys, pk)
            idx = jnp.where(take_self, idx, pi)
            j //= 2
        k *= 2
    return keys, idx


def _sort_finalize_body(ckeys_ref, cidx_ref, comp_ref,
                        out_idx_ref, out_w_ref, out_mask_ref):
    keys, idx = _bitonic_sort_desc(ckeys_ref[...], cidx_ref[...])
    ts = _key_to_f32(keys)
    m = ts[:, 0:1]
    e = jnp.exp(ts - m)
    w = e / jnp.sum(e, axis=1, keepdims=True)
    comp = comp_ref[...]
    budgets = jnp.round(
        jnp.clip(MIN_P + (MAX_P - MIN_P) * comp * comp, MIN_P, MAX_P)
    ).astype(jnp.int32)
    pos = jax.lax.broadcasted_iota(jnp.int32, out_mask_ref.shape, 1)
    mask = (pos < budgets).astype(jnp.float32)
    out_idx_ref[...] = idx
    out_w_ref[...] = w * mask
    out_mask_ref[...] = mask


def kernel(x, W1, b1, W2, b2, S1, bs1, S2, bs2):
    n_ra = TOKENS // ROW_A
    hs, comp = pl.pallas_call(
        _hs_complexity_body,
        grid=(n_ra,),
        in_specs=[
            pl.BlockSpec((ROW_A, INPUT_DIM), lambda i: (i, 0)),
            pl.BlockSpec((INPUT_DIM, 128), lambda i: (0, 0)),
            pl.BlockSpec((128,), lambda i: (0,)),
            pl.BlockSpec((128, 1), lambda i: (0, 0)),
            pl.BlockSpec((1,), lambda i: (0,)),
            pl.BlockSpec((INPUT_DIM, HIDDEN_DIM), lambda i: (0, 0)),
            pl.BlockSpec((HIDDEN_DIM,), lambda i: (0,)),
        ],
        out_specs=[
            pl.BlockSpec((ROW_A, HIDDEN_DIM), lambda i: (i, 0)),
            pl.BlockSpec((ROW_A, 1), lambda i: (i, 0)),
        ],
        out_shape=[
            jax.ShapeDtypeStruct((TOKENS, HIDDEN_DIM), jnp.float32),
            jax.ShapeDtypeStruct((TOKENS, 1), jnp.float32),
        ],
    )(x, W1, b1, W2, b2, S1, bs1)

    n_rb = TOKENS // ROW_B
    keys, t_b, quota_b = pl.pallas_call(
        _keys_bisect_body,
        grid=(n_rb,),
        in_specs=[
            pl.BlockSpec((ROW_B, HIDDEN_DIM), lambda i: (i, 0)),
            pl.BlockSpec((HIDDEN_DIM, POOL_SIZE), lambda i: (0, 0)),
            pl.BlockSpec((POOL_SIZE,), lambda i: (0,)),
        ],
        out_specs=[
            pl.BlockSpec((ROW_B, POOL_SIZE), lambda i: (i, 0)),
            pl.BlockSpec((ROW_B, 16), lambda i: (i, 0)),
            pl.BlockSpec((ROW_B, 16), lambda i: (i, 0)),
        ],
        out_shape=[
            jax.ShapeDtypeStruct((TOKENS, POOL_SIZE), jnp.int32),
            jax.ShapeDtypeStruct((TOKENS, 16), jnp.int32),
            jax.ShapeDtypeStruct((TOKENS, 16), jnp.int32),
        ],
        scratch_shapes=[
            pltpu.VMEM((ROW_B, 1), jnp.int32),
            pltpu.VMEM((ROW_B, 1), jnp.int32),
            pltpu.VMEM((ROW_B, 1), jnp.int32),
        ],
    )(hs, S2, bs2)

    ckeys, cidx = _compact_call(keys, t_b, quota_b)

    n_rd = TOKENS // ROW_D
    out_idx, out_w, out_mask = pl.pallas_call(
        _sort_finalize_body,
        grid=(n_rd,),
        in_specs=[
            pl.BlockSpec((ROW_D, K), lambda i: (i, 0)),
            pl.BlockSpec((ROW_D, K), lambda i: (i, 0)),
            pl.BlockSpec((ROW_D, 1), lambda i: (i, 0)),
        ],
        out_specs=[
            pl.BlockSpec((ROW_D, K), lambda i: (i, 0)),
            pl.BlockSpec((ROW_D, K), lambda i: (i, 0)),
            pl.BlockSpec((ROW_D, K), lambda i: (i, 0)),
        ],
        out_shape=[
            jax.ShapeDtypeStruct((TOKENS, K), jnp.int32),
            jax.ShapeDtypeStruct((TOKENS, K), jnp.float32),
            jax.ShapeDtypeStruct((TOKENS, K), jnp.float32),
        ],
    )(ckeys, cidx, comp)
    return out_idx, out_w, out_mask, comp


# speedup vs baseline: 1.5994x; 1.0611x over previous
import functools
import jax
import jax.numpy as jnp
from jax import lax
from jax.experimental import pallas as pl
from jax.experimental.pallas import tpu as pltpu
from jax.experimental.pallas import tpu_sc as plsc

TOKENS = 8192
INPUT_DIM = 1024
HIDDEN_DIM = 256
POOL_SIZE = 16384
K = 1024
MIN_P = 100.0
MAX_P = 1024.0

ROW_A = 256
ROW_B = 128
ROW_D = 128
MAXI32 = 0x7FFFFFFF


def _hs_complexity_body(x_ref, W1_ref, b1_ref, W2_ref, b2_ref, S1_ref, bs1_ref,
                        hs_ref, comp_ref):
    x = x_ref[...]
    h = jnp.maximum(jnp.dot(x, W1_ref[...], preferred_element_type=jnp.float32)
                    + b1_ref[...], 0.0)
    logit = jnp.dot(h, W2_ref[...], preferred_element_type=jnp.float32) + b2_ref[...]
    comp_ref[...] = jax.nn.sigmoid(logit)
    hs_ref[...] = jnp.maximum(
        jnp.dot(x, S1_ref[...], preferred_element_type=jnp.float32) + bs1_ref[...], 0.0)


def _f32_to_key(s):
    b = jax.lax.bitcast_convert_type(s, jnp.int32)
    return jnp.where(b >= 0, b, MAXI32 - b)


def _key_to_f32(k):
    b = jnp.where(k >= 0, k, MAXI32 - k)
    return jax.lax.bitcast_convert_type(b, jnp.float32)


def _keys_bisect_body(hs_ref, S2_ref, bs2_ref, keys_ref, t_ref, quota_ref,
                      lo_ref, hi_ref, fr_ref):
    scores = (jnp.dot(hs_ref[...], S2_ref[...],
                      preferred_element_type=jnp.float32) + bs2_ref[...])
    keys = _f32_to_key(scores)
    keys_ref[...] = keys

    lo0 = jnp.min(keys, axis=1, keepdims=True)
    hi0 = jnp.max(keys, axis=1, keepdims=True) + 1

    mu = jnp.mean(scores, axis=1, keepdims=True)
    sd = jnp.sqrt(jnp.maximum(
        jnp.mean(scores * scores, axis=1, keepdims=True) - mu * mu, 0.0))
    c1 = _f32_to_key(mu + 1.1 * sd)
    c2 = _f32_to_key(mu + 2.0 * sd)
    n1 = jnp.sum((keys >= c1).astype(jnp.int32), axis=1, keepdims=True)
    n2 = jnp.sum((keys >= c2).astype(jnp.int32), axis=1, keepdims=True)
    lo = jnp.where(n1 >= K, c1, lo0)
    lo = jnp.where(n2 >= K, jnp.maximum(c2, lo), lo)
    hi = jnp.where(n2 < K, c2, hi0)
    hi = jnp.where(n1 < K, jnp.minimum(c1, hi), hi)
    fr0 = jnp.logical_or(n1 == K, n2 == K).astype(jnp.int32)
    lo = jnp.where(n1 == K, c1, lo)
    lo = jnp.where(n2 == K, c2, lo)
    lo_ref[...] = lo
    hi_ref[...] = hi
    fr_ref[...] = fr0

    def cond(carry):
        it, done = carry
        return jnp.logical_and(it < 34, done == 0)

    def body(carry):
        it, done = carry
        lo = lo_ref[...]
        hi = hi_ref[...]
        fr = fr_ref[...]
        mid = (lo >> 1) + (hi >> 1) + (lo & hi & 1)
        cnt = jnp.sum((keys >= mid).astype(jnp.int32), axis=1, keepdims=True)
        active = (fr == 0) & (hi - lo != 1)
        ge = cnt >= K
        new_lo = jnp.where(active & ge, mid, lo)
        new_hi = jnp.where(active & jnp.logical_not(ge), mid, hi)
        new_fr = fr | (active & (cnt == K)).astype(jnp.int32)
        lo_ref[...] = new_lo
        hi_ref[...] = new_hi
        fr_ref[...] = new_fr
        all_done = jnp.all((new_fr != 0) | (new_hi - new_lo == 1))
        return it + 1, all_done.astype(jnp.int32)

    jax.lax.while_loop(cond, body, (jnp.int32(0), jnp.int32(0)))

    lo = lo_ref[...]
    frozen = fr_ref[...] != 0
    sel = keys >= lo
    minsel = jnp.min(jnp.where(sel, keys, MAXI32), axis=1, keepdims=True)
    t = jnp.where(frozen, minsel, lo)
    cnt_gt = jnp.sum((keys > t).astype(jnp.int32), axis=1, keepdims=True)
    quota = K - cnt_gt
    t_ref[...] = jnp.broadcast_to(t, t_ref.shape)
    quota_ref[...] = jnp.broadcast_to(quota, quota_ref.shape)


NW = 32
RPW = TOKENS // NW
OPAD = K + 16


GROUP = 8


def _compact_body(keys_hbm, t_hbm, q_hbm, ck_hbm, ci_hbm,
                  kbuf0, kbuf1, tbuf, qbuf, outk, outi, insem):
    c = lax.axis_index("c")
    s = lax.axis_index("s")
    w = s * 2 + c
    row0 = w * RPW
    iota = lax.iota(jnp.int32, 16)
    pltpu.sync_copy(t_hbm.at[pl.ds(row0 * 16, RPW * 16)], tbuf)
    pltpu.sync_copy(q_hbm.at[pl.ds(row0 * 16, RPW * 16)], qbuf)
    kbufs = (kbuf0, kbuf1)
    pltpu.make_async_copy(keys_hbm.at[row0], kbuf0, insem.at[0]).start()

    zero16 = jnp.zeros((16,), jnp.int32)

    def gbody(g, _unused):
        for rr in range(GROUP):
            m = g * GROUP + rr
            p = rr % 2
            np_ = (rr + 1) % 2
            row = row0 + m
            nxt = row0 + lax.rem(m + 1, RPW)
            pltpu.make_async_copy(keys_hbm.at[nxt], kbufs[np_],
                                  insem.at[np_]).start()
            pltpu.make_async_copy(keys_hbm.at[row], kbufs[p],
                                  insem.at[p]).wait()
            t = tbuf[pl.ds(m * 16, 16)]
            q = qbuf[pl.ds(m * 16, 16)]
            krow = kbufs[p]

            def vbody(j, carry):
                acc, eqb = carry
                v = krow[pl.ds(j * 16, 16)]
                m_gt = v > t
                m_eq = v == t
                c2 = plsc.cumsum(m_gt.astype(jnp.int32) * 65536
                                 + m_eq.astype(jnp.int32))
                cgt = c2 >> 16
                ceq = c2 & 0xFFFF
                rank = ceq + eqb
                m_sel = m_gt | (m_eq & (rank <= q))
                keep_incl = jnp.minimum(rank, q) - jnp.minimum(eqb, q)
                pos = (acc + cgt + keep_incl - m_sel.astype(jnp.int32)
                       + (rr * K))
                plsc.store_scatter(outk, [pos], v, mask=m_sel)
                plsc.store_scatter(outi, [pos], iota + j * 16, mask=m_sel)
                return (acc + plsc.all_reduce_population_count(m_sel),
                        eqb + plsc.all_reduce_population_count(m_eq))

            plsc.parallel_loop(0, POOL_SIZE // 16, 1, unroll=4,
                               carry=(zero16, zero16))(vbody)
        base = (row0 + g * GROUP) * K
        pltpu.sync_copy(outk, ck_hbm.at[pl.ds(base, GROUP * K)])
        pltpu.sync_copy(outi, ci_hbm.at[pl.ds(base, GROUP * K)])
        return _unused

    lax.fori_loop(0, RPW // GROUP, gbody, jnp.int32(0))
    pltpu.make_async_copy(keys_hbm.at[row0], kbuf0, insem.at[0]).wait()


def _compact_call(keys, t_b, quota_b):
    mesh = plsc.VectorSubcoreMesh(core_axis_name="c", subcore_axis_name="s", num_cores=2, num_subcores=16)
    f = pl.kernel(
        _compact_body,
        out_type=[
            jax.ShapeDtypeStruct((TOKENS * K,), jnp.int32),
            jax.ShapeDtypeStruct((TOKENS * K,), jnp.int32),
        ],
        mesh=mesh,
        compiler_params=pltpu.CompilerParams(needs_layout_passes=False),
        scratch_types=[
            pltpu.VMEM((POOL_SIZE,), jnp.int32),
            pltpu.VMEM((POOL_SIZE,), jnp.int32),
            pltpu.VMEM((RPW * 16,), jnp.int32),
            pltpu.VMEM((RPW * 16,), jnp.int32),
            pltpu.VMEM((GROUP * K,), jnp.int32),
            pltpu.VMEM((GROUP * K,), jnp.int32),
            pltpu.SemaphoreType.DMA((2,)),
        ],
    )
    ck, ci = f(keys, t_b.reshape(-1), quota_b.reshape(-1))
    return ck.reshape(TOKENS, K), ci.reshape(TOKENS, K)


def _bitonic_sort_desc(keys, idx):
    R, N = keys.shape
    lane = jax.lax.broadcasted_iota(jnp.int32, (R, N), 1)
    k = 2
    while k <= N:
        j = k // 2
        while j >= 1:
            low = (lane & j) == 0
            pk = jnp.where(low, pltpu.roll(keys, N - j, 1), pltpu.roll(keys, j, 1))
            pi = jnp.where(low, pltpu.roll(idx, N - j, 1), pltpu.roll(idx, j, 1))
            self_beats = (keys > pk) | ((keys == pk) & (idx < pi))
            want_max = ((lane & k) == 0) == low
            take_self = want_max == self_beats
            keys = jnp.where(take_self, keys, pk)
            idx = jnp.where(take_self, idx, pi)
            j //= 2
        k *= 2
    return keys, idx


def _sort_finalize_body(ckeys_ref, cidx_ref, comp_ref,
                        out_idx_ref, out_w_ref, out_mask_ref):
    keys, idx = _bitonic_sort_desc(ckeys_ref[...], cidx_ref[...])
    ts = _key_to_f32(keys)
    m = ts[:, 0:1]
    e = jnp.exp(ts - m)
    w = e / jnp.sum(e, axis=1, keepdims=True)
    comp = comp_ref[...]
    budgets = jnp.round(
        jnp.clip(MIN_P + (MAX_P - MIN_P) * comp * comp, MIN_P, MAX_P)
    ).astype(jnp.int32)
    pos = jax.lax.broadcasted_iota(jnp.int32, out_mask_ref.shape, 1)
    mask = (pos < budgets).astype(jnp.float32)
    out_idx_ref[...] = idx
    out_w_ref[...] = w * mask
    out_mask_ref[...] = mask


def kernel(x, W1, b1, W2, b2, S1, bs1, S2, bs2):
    n_ra = TOKENS // ROW_A
    hs, comp = pl.pallas_call(
        _hs_complexity_body,
        grid=(n_ra,),
        in_specs=[
            pl.BlockSpec((ROW_A, INPUT_DIM), lambda i: (i, 0)),
            pl.BlockSpec((INPUT_DIM, 128), lambda i: (0, 0)),
            pl.BlockSpec((128,), lambda i: (0,)),
            pl.BlockSpec((128, 1), lambda i: (0, 0)),
            pl.BlockSpec((1,), lambda i: (0,)),
            pl.BlockSpec((INPUT_DIM, HIDDEN_DIM), lambda i: (0, 0)),
            pl.BlockSpec((HIDDEN_DIM,), lambda i: (0,)),
        ],
        out_specs=[
            pl.BlockSpec((ROW_A, HIDDEN_DIM), lambda i: (i, 0)),
            pl.BlockSpec((ROW_A, 1), lambda i: (i, 0)),
        ],
        out_shape=[
            jax.ShapeDtypeStruct((TOKENS, HIDDEN_DIM), jnp.float32),
            jax.ShapeDtypeStruct((TOKENS, 1), jnp.float32),
        ],
    )(x, W1, b1, W2, b2, S1, bs1)

    n_rb = TOKENS // ROW_B
    keys, t_b, quota_b = pl.pallas_call(
        _keys_bisect_body,
        grid=(n_rb,),
        in_specs=[
            pl.BlockSpec((ROW_B, HIDDEN_DIM), lambda i: (i, 0)),
            pl.BlockSpec((HIDDEN_DIM, POOL_SIZE), lambda i: (0, 0)),
            pl.BlockSpec((POOL_SIZE,), lambda i: (0,)),
        ],
        out_specs=[
            pl.BlockSpec((ROW_B, POOL_SIZE), lambda i: (i, 0)),
            pl.BlockSpec((ROW_B, 16), lambda i: (i, 0)),
            pl.BlockSpec((ROW_B, 16), lambda i: (i, 0)),
        ],
        out_shape=[
            jax.ShapeDtypeStruct((TOKENS, POOL_SIZE), jnp.int32),
            jax.ShapeDtypeStruct((TOKENS, 16), jnp.int32),
            jax.ShapeDtypeStruct((TOKENS, 16), jnp.int32),
        ],
        scratch_shapes=[
            pltpu.VMEM((ROW_B, 1), jnp.int32),
            pltpu.VMEM((ROW_B, 1), jnp.int32),
            pltpu.VMEM((ROW_B, 1), jnp.int32),
        ],
    )(hs, S2, bs2)

    ckeys, cidx = _compact_call(keys, t_b, quota_b)

    n_rd = TOKENS // ROW_D
    out_idx, out_w, out_mask = pl.pallas_call(
        _sort_finalize_body,
        grid=(n_rd,),
        in_specs=[
            pl.BlockSpec((ROW_D, K), lambda i: (i, 0)),
            pl.BlockSpec((ROW_D, K), lambda i: (i, 0)),
            pl.BlockSpec((ROW_D, 1), lambda i: (i, 0)),
        ],
        out_specs=[
            pl.BlockSpec((ROW_D, K), lambda i: (i, 0)),
            pl.BlockSpec((ROW_D, K), lambda i: (i, 0)),
            pl.BlockSpec((ROW_D, K), lambda i: (i, 0)),
        ],
        out_shape=[
            jax.ShapeDtypeStruct((TOKENS, K), jnp.int32),
            jax.ShapeDtypeStruct((TOKENS, K), jnp.float32),
            jax.ShapeDtypeStruct((TOKENS, K), jnp.float32),
        ],
    )(ckeys, cidx, comp)
    return out_idx, out_w, out_mask, comp


# speedup vs baseline: 1.7369x; 1.0859x over previous
import functools
import jax
import jax.numpy as jnp
from jax import lax
from jax.experimental import pallas as pl
from jax.experimental.pallas import tpu as pltpu
from jax.experimental.pallas import tpu_sc as plsc

TOKENS = 8192
INPUT_DIM = 1024
HIDDEN_DIM = 256
POOL_SIZE = 16384
K = 1024
MIN_P = 100.0
MAX_P = 1024.0

ROW_A = 256
ROW_B = 128
ROW_D = 128
MAXI32 = 0x7FFFFFFF


def _hs_complexity_body(x_ref, W1_ref, b1_ref, W2_ref, b2_ref, S1_ref, bs1_ref,
                        hs_ref, comp_ref):
    x = x_ref[...]
    h = jnp.maximum(jnp.dot(x, W1_ref[...], preferred_element_type=jnp.float32)
                    + b1_ref[...], 0.0)
    logit = jnp.dot(h, W2_ref[...], preferred_element_type=jnp.float32) + b2_ref[...]
    comp_ref[...] = jax.nn.sigmoid(logit)
    hs_ref[...] = jnp.maximum(
        jnp.dot(x, S1_ref[...], preferred_element_type=jnp.float32) + bs1_ref[...], 0.0)


def _f32_to_key(s):
    b = jax.lax.bitcast_convert_type(s, jnp.int32)
    return jnp.where(b >= 0, b, MAXI32 - b)


def _key_to_f32(k):
    b = jnp.where(k >= 0, k, MAXI32 - k)
    return jax.lax.bitcast_convert_type(b, jnp.float32)


def _keys_bisect_body(hs_ref, S2_ref, bs2_ref, keys_ref, t_ref, quota_ref,
                      lo_ref, hi_ref, fr_ref):
    scores = (jnp.dot(hs_ref[...], S2_ref[...],
                      preferred_element_type=jnp.float32) + bs2_ref[...])
    keys = _f32_to_key(scores)
    keys_ref[...] = keys

    lo0 = jnp.min(keys, axis=1, keepdims=True)
    hi0 = jnp.max(keys, axis=1, keepdims=True) + 1

    mu = jnp.mean(scores, axis=1, keepdims=True)
    sd = jnp.sqrt(jnp.maximum(
        jnp.mean(scores * scores, axis=1, keepdims=True) - mu * mu, 0.0))
    c1 = _f32_to_key(mu + 1.1 * sd)
    c2 = _f32_to_key(mu + 2.0 * sd)
    n1 = jnp.sum((keys >= c1).astype(jnp.int32), axis=1, keepdims=True)
    n2 = jnp.sum((keys >= c2).astype(jnp.int32), axis=1, keepdims=True)
    lo = jnp.where(n1 >= K, c1, lo0)
    lo = jnp.where(n2 >= K, jnp.maximum(c2, lo), lo)
    hi = jnp.where(n2 < K, c2, hi0)
    hi = jnp.where(n1 < K, jnp.minimum(c1, hi), hi)
    fr0 = jnp.logical_or(n1 == K, n2 == K).astype(jnp.int32)
    lo = jnp.where(n1 == K, c1, lo)
    lo = jnp.where(n2 == K, c2, lo)
    lo_ref[...] = lo
    hi_ref[...] = hi
    fr_ref[...] = fr0

    def cond(carry):
        it, done = carry
        return jnp.logical_and(it < 34, done == 0)

    def body(carry):
        it, done = carry
        lo = lo_ref[...]
        hi = hi_ref[...]
        fr = fr_ref[...]
        mid = (lo >> 1) + (hi >> 1) + (lo & hi & 1)
        cnt = jnp.sum((keys >= mid).astype(jnp.int32), axis=1, keepdims=True)
        active = (fr == 0) & (hi - lo != 1)
        ge = cnt >= K
        new_lo = jnp.where(active & ge, mid, lo)
        new_hi = jnp.where(active & jnp.logical_not(ge), mid, hi)
        new_fr = fr | (active & (cnt == K)).astype(jnp.int32)
        lo_ref[...] = new_lo
        hi_ref[...] = new_hi
        fr_ref[...] = new_fr
        all_done = jnp.all((new_fr != 0) | (new_hi - new_lo == 1))
        return it + 1, all_done.astype(jnp.int32)

    jax.lax.while_loop(cond, body, (jnp.int32(0), jnp.int32(0)))

    lo = lo_ref[...]
    frozen = fr_ref[...] != 0
    sel = keys >= lo
    minsel = jnp.min(jnp.where(sel, keys, MAXI32), axis=1, keepdims=True)
    t = jnp.where(frozen, minsel, lo)
    cnt_gt = jnp.sum((keys > t).astype(jnp.int32), axis=1, keepdims=True)
    quota = K - cnt_gt
    t_ref[...] = jnp.broadcast_to(t, t_ref.shape)
    quota_ref[...] = jnp.broadcast_to(quota, quota_ref.shape)


NW = 32
RPW = TOKENS // NW
OPAD = K + 16


GROUP = 8


def _make_compact_body(rpw):
  def _compact_body(keys_hbm, t_hbm, q_hbm, ck_hbm, ci_hbm,
                    kbuf0, kbuf1, tbuf, qbuf, outk, outi, insem):
    c = lax.axis_index("c")
    s = lax.axis_index("s")
    w = s * 2 + c
    row0 = w * rpw
    iota = lax.iota(jnp.int32, 16)
    pltpu.sync_copy(t_hbm.at[pl.ds(row0 * 16, rpw * 16)], tbuf)
    pltpu.sync_copy(q_hbm.at[pl.ds(row0 * 16, rpw * 16)], qbuf)
    kbufs = (kbuf0, kbuf1)
    pltpu.make_async_copy(keys_hbm.at[row0], kbuf0, insem.at[0]).start()

    zero16 = jnp.zeros((16,), jnp.int32)

    def gbody(g, _unused):
        for rr in range(GROUP):
            m = g * GROUP + rr
            p = rr % 2
            np_ = (rr + 1) % 2
            row = row0 + m
            nxt = row0 + lax.rem(m + 1, rpw)
            pltpu.make_async_copy(keys_hbm.at[nxt], kbufs[np_],
                                  insem.at[np_]).start()
            pltpu.make_async_copy(keys_hbm.at[row], kbufs[p],
                                  insem.at[p]).wait()
            t = tbuf[pl.ds(m * 16, 16)]
            q = qbuf[pl.ds(m * 16, 16)]
            krow = kbufs[p]

            def vbody(j, carry):
                acc, eqb = carry
                v = krow[pl.ds(j * 16, 16)]
                m_gt = v > t
                m_eq = v == t
                c2 = plsc.cumsum(m_gt.astype(jnp.int32) * 65536
                                 + m_eq.astype(jnp.int32))
                cgt = c2 >> 16
                ceq = c2 & 0xFFFF
                rank = ceq + eqb
                m_sel = m_gt | (m_eq & (rank <= q))
                keep_incl = jnp.minimum(rank, q) - jnp.minimum(eqb, q)
                pos = (acc + cgt + keep_incl - m_sel.astype(jnp.int32)
                       + (rr * K))
                plsc.store_scatter(outk, [pos], v, mask=m_sel)
                plsc.store_scatter(outi, [pos], iota + j * 16, mask=m_sel)
                return (acc + plsc.all_reduce_population_count(m_sel),
                        eqb + plsc.all_reduce_population_count(m_eq))

            plsc.parallel_loop(0, POOL_SIZE // 16, 1, unroll=4,
                               carry=(zero16, zero16))(vbody)
        base = (row0 + g * GROUP) * K
        pltpu.sync_copy(outk, ck_hbm.at[pl.ds(base, GROUP * K)])
        pltpu.sync_copy(outi, ci_hbm.at[pl.ds(base, GROUP * K)])
        return _unused

    lax.fori_loop(0, rpw // GROUP, gbody, jnp.int32(0))
    pltpu.make_async_copy(keys_hbm.at[row0], kbuf0, insem.at[0]).wait()

  return _compact_body


def _compact_call(keys, t_b, quota_b):
    tokens = keys.shape[0]
    rpw = tokens // NW
    mesh = plsc.VectorSubcoreMesh(core_axis_name="c", subcore_axis_name="s", num_cores=2, num_subcores=16)
    f = pl.kernel(
        _make_compact_body(rpw),
        out_type=[
            jax.ShapeDtypeStruct((tokens * K,), jnp.int32),
            jax.ShapeDtypeStruct((tokens * K,), jnp.int32),
        ],
        mesh=mesh,
        compiler_params=pltpu.CompilerParams(needs_layout_passes=False),
        scratch_types=[
            pltpu.VMEM((POOL_SIZE,), jnp.int32),
            pltpu.VMEM((POOL_SIZE,), jnp.int32),
            pltpu.VMEM((rpw * 16,), jnp.int32),
            pltpu.VMEM((rpw * 16,), jnp.int32),
            pltpu.VMEM((GROUP * K,), jnp.int32),
            pltpu.VMEM((GROUP * K,), jnp.int32),
            pltpu.SemaphoreType.DMA((2,)),
        ],
    )
    ck, ci = f(keys, t_b.reshape(-1), quota_b.reshape(-1))
    return ck.reshape(tokens, K), ci.reshape(tokens, K)


def _bitonic_sort_desc(keys, idx):
    R, N = keys.shape
    lane = jax.lax.broadcasted_iota(jnp.int32, (R, N), 1)
    k = 2
    while k <= N:
        j = k // 2
        while j >= 1:
            low = (lane & j) == 0
            pk = jnp.where(low, pltpu.roll(keys, N - j, 1), pltpu.roll(keys, j, 1))
            pi = jnp.where(low, pltpu.roll(idx, N - j, 1), pltpu.roll(idx, j, 1))
            self_beats = (keys > pk) | ((keys == pk) & (idx < pi))
            want_max = ((lane & k) == 0) == low
            take_self = want_max == self_beats
            keys = jnp.where(take_self, keys, pk)
            idx = jnp.where(take_self, idx, pi)
            j //= 2
        k *= 2
    return keys, idx


def _sort_finalize_body(ckeys_ref, cidx_ref, comp_ref,
                        out_idx_ref, out_w_ref, out_mask_ref):
    keys, idx = _bitonic_sort_desc(ckeys_ref[...], cidx_ref[...])
    ts = _key_to_f32(keys)
    m = ts[:, 0:1]
    e = jnp.exp(ts - m)
    w = e / jnp.sum(e, axis=1, keepdims=True)
    comp = comp_ref[...]
    budgets = jnp.round(
        jnp.clip(MIN_P + (MAX_P - MIN_P) * comp * comp, MIN_P, MAX_P)
    ).astype(jnp.int32)
    pos = jax.lax.broadcasted_iota(jnp.int32, out_mask_ref.shape, 1)
    mask = (pos < budgets).astype(jnp.float32)
    out_idx_ref[...] = idx
    out_w_ref[...] = w * mask
    out_mask_ref[...] = mask


def _run_chunk(x, W1, b1, W2, b2, S1, bs1, S2, bs2):
    tokens = x.shape[0]
    n_ra = tokens // ROW_A
    hs, comp = pl.pallas_call(
        _hs_complexity_body,
        grid=(n_ra,),
        in_specs=[
            pl.BlockSpec((ROW_A, INPUT_DIM), lambda i: (i, 0)),
            pl.BlockSpec((INPUT_DIM, 128), lambda i: (0, 0)),
            pl.BlockSpec((128,), lambda i: (0,)),
            pl.BlockSpec((128, 1), lambda i: (0, 0)),
            pl.BlockSpec((1,), lambda i: (0,)),
            pl.BlockSpec((INPUT_DIM, HIDDEN_DIM), lambda i: (0, 0)),
            pl.BlockSpec((HIDDEN_DIM,), lambda i: (0,)),
        ],
        out_specs=[
            pl.BlockSpec((ROW_A, HIDDEN_DIM), lambda i: (i, 0)),
            pl.BlockSpec((ROW_A, 1), lambda i: (i, 0)),
        ],
        out_shape=[
            jax.ShapeDtypeStruct((tokens, HIDDEN_DIM), jnp.float32),
            jax.ShapeDtypeStruct((tokens, 1), jnp.float32),
        ],
    )(x, W1, b1, W2, b2, S1, bs1)

    n_rb = tokens // ROW_B
    keys, t_b, quota_b = pl.pallas_call(
        _keys_bisect_body,
        grid=(n_rb,),
        in_specs=[
            pl.BlockSpec((ROW_B, HIDDEN_DIM), lambda i: (i, 0)),
            pl.BlockSpec((HIDDEN_DIM, POOL_SIZE), lambda i: (0, 0)),
            pl.BlockSpec((POOL_SIZE,), lambda i: (0,)),
        ],
        out_specs=[
            pl.BlockSpec((ROW_B, POOL_SIZE), lambda i: (i, 0)),
            pl.BlockSpec((ROW_B, 16), lambda i: (i, 0)),
            pl.BlockSpec((ROW_B, 16), lambda i: (i, 0)),
        ],
        out_shape=[
            jax.ShapeDtypeStruct((tokens, POOL_SIZE), jnp.int32),
            jax.ShapeDtypeStruct((tokens, 16), jnp.int32),
            jax.ShapeDtypeStruct((tokens, 16), jnp.int32),
        ],
        scratch_shapes=[
            pltpu.VMEM((ROW_B, 1), jnp.int32),
            pltpu.VMEM((ROW_B, 1), jnp.int32),
            pltpu.VMEM((ROW_B, 1), jnp.int32),
        ],
    )(hs, S2, bs2)

    ckeys, cidx = _compact_call(keys, t_b, quota_b)

    n_rd = tokens // ROW_D
    out_idx, out_w, out_mask = pl.pallas_call(
        _sort_finalize_body,
        grid=(n_rd,),
        in_specs=[
            pl.BlockSpec((ROW_D, K), lambda i: (i, 0)),
            pl.BlockSpec((ROW_D, K), lambda i: (i, 0)),
            pl.BlockSpec((ROW_D, 1), lambda i: (i, 0)),
        ],
        out_specs=[
            pl.BlockSpec((ROW_D, K), lambda i: (i, 0)),
            pl.BlockSpec((ROW_D, K), lambda i: (i, 0)),
            pl.BlockSpec((ROW_D, K), lambda i: (i, 0)),
        ],
        out_shape=[
            jax.ShapeDtypeStruct((tokens, K), jnp.int32),
            jax.ShapeDtypeStruct((tokens, K), jnp.float32),
            jax.ShapeDtypeStruct((tokens, K), jnp.float32),
        ],
    )(ckeys, cidx, comp)
    return out_idx, out_w, out_mask, comp


CHUNKS = 2


def kernel(x, W1, b1, W2, b2, S1, bs1, S2, bs2):
    tc = TOKENS // CHUNKS
    parts = [
        _run_chunk(x[i * tc:(i + 1) * tc], W1, b1, W2, b2, S1, bs1, S2, bs2)
        for i in range(CHUNKS)
    ]
    return tuple(jnp.concatenate(leaves, axis=0) for leaves in zip(*parts))


# speedup vs baseline: 1.8062x; 1.0399x over previous
import functools
import jax
import jax.numpy as jnp
from jax import lax
from jax.experimental import pallas as pl
from jax.experimental.pallas import tpu as pltpu
from jax.experimental.pallas import tpu_sc as plsc

TOKENS = 8192
INPUT_DIM = 1024
HIDDEN_DIM = 256
POOL_SIZE = 16384
K = 1024
MIN_P = 100.0
MAX_P = 1024.0

ROW_A = 256
ROW_B = 128
ROW_D = 128
MAXI32 = 0x7FFFFFFF


def _hs_complexity_body(x_ref, W1_ref, b1_ref, W2_ref, b2_ref, S1_ref, bs1_ref,
                        hs_ref, comp_ref):
    x = x_ref[...]
    h = jnp.maximum(jnp.dot(x, W1_ref[...], preferred_element_type=jnp.float32)
                    + b1_ref[...], 0.0)
    logit = jnp.dot(h, W2_ref[...], preferred_element_type=jnp.float32) + b2_ref[...]
    comp_ref[...] = jax.nn.sigmoid(logit)
    hs_ref[...] = jnp.maximum(
        jnp.dot(x, S1_ref[...], preferred_element_type=jnp.float32) + bs1_ref[...], 0.0)


def _f32_to_key(s):
    b = jax.lax.bitcast_convert_type(s, jnp.int32)
    return jnp.where(b >= 0, b, MAXI32 - b)


def _key_to_f32(k):
    b = jnp.where(k >= 0, k, MAXI32 - k)
    return jax.lax.bitcast_convert_type(b, jnp.float32)


def _keys_bisect_body(hs_ref, S2_ref, bs2_ref, keys_ref, t_ref, quota_ref,
                      lo_ref, hi_ref, fr_ref):
    scores = (jnp.dot(hs_ref[...], S2_ref[...],
                      preferred_element_type=jnp.float32) + bs2_ref[...])
    keys = _f32_to_key(scores)
    keys_ref[...] = keys

    lo0 = jnp.min(keys, axis=1, keepdims=True)
    hi0 = jnp.max(keys, axis=1, keepdims=True) + 1

    mu = jnp.mean(scores, axis=1, keepdims=True)
    sd = jnp.sqrt(jnp.maximum(
        jnp.mean(scores * scores, axis=1, keepdims=True) - mu * mu, 0.0))
    c1 = _f32_to_key(mu + 1.1 * sd)
    c2 = _f32_to_key(mu + 2.0 * sd)
    n1 = jnp.sum((keys >= c1).astype(jnp.int32), axis=1, keepdims=True)
    n2 = jnp.sum((keys >= c2).astype(jnp.int32), axis=1, keepdims=True)
    lo = jnp.where(n1 >= K, c1, lo0)
    lo = jnp.where(n2 >= K, jnp.maximum(c2, lo), lo)
    hi = jnp.where(n2 < K, c2, hi0)
    hi = jnp.where(n1 < K, jnp.minimum(c1, hi), hi)
    fr0 = jnp.logical_or(n1 == K, n2 == K).astype(jnp.int32)
    lo = jnp.where(n1 == K, c1, lo)
    lo = jnp.where(n2 == K, c2, lo)
    lo_ref[...] = lo
    hi_ref[...] = hi
    fr_ref[...] = fr0

    def cond(carry):
        it, done = carry
        return jnp.logical_and(it < 34, done == 0)

    def body(carry):
        it, done = carry
        lo = lo_ref[...]
        hi = hi_ref[...]
        fr = fr_ref[...]
        mid = (lo >> 1) + (hi >> 1) + (lo & hi & 1)
        cnt = jnp.sum((keys >= mid).astype(jnp.int32), axis=1, keepdims=True)
        active = (fr == 0) & (hi - lo != 1)
        ge = cnt >= K
        new_lo = jnp.where(active & ge, mid, lo)
        new_hi = jnp.where(active & jnp.logical_not(ge), mid, hi)
        new_fr = fr | (active & (cnt == K)).astype(jnp.int32)
        lo_ref[...] = new_lo
        hi_ref[...] = new_hi
        fr_ref[...] = new_fr
        all_done = jnp.all((new_fr != 0) | (new_hi - new_lo == 1))
        return it + 1, all_done.astype(jnp.int32)

    jax.lax.while_loop(cond, body, (jnp.int32(0), jnp.int32(0)))

    lo = lo_ref[...]
    frozen = fr_ref[...] != 0
    sel = keys >= lo
    minsel = jnp.min(jnp.where(sel, keys, MAXI32), axis=1, keepdims=True)
    t = jnp.where(frozen, minsel, lo)
    cnt_gt = jnp.sum((keys > t).astype(jnp.int32), axis=1, keepdims=True)
    quota = K - cnt_gt
    t_ref[...] = jnp.broadcast_to(t, t_ref.shape)
    quota_ref[...] = jnp.broadcast_to(quota, quota_ref.shape)


NW = 32
RPW = TOKENS // NW
OPAD = K + 16


GROUP = 8


def _make_compact_body(rpw):
  def _compact_body(keys_hbm, t_hbm, q_hbm, ck_hbm, ci_hbm,
                    kbuf0, kbuf1, tbuf, qbuf, outk, outi, insem):
    c = lax.axis_index("c")
    s = lax.axis_index("s")
    w = s * 2 + c
    row0 = w * rpw
    iota = lax.iota(jnp.int32, 16)
    pltpu.sync_copy(t_hbm.at[pl.ds(row0 * 16, rpw * 16)], tbuf)
    pltpu.sync_copy(q_hbm.at[pl.ds(row0 * 16, rpw * 16)], qbuf)
    kbufs = (kbuf0, kbuf1)
    pltpu.make_async_copy(keys_hbm.at[row0], kbuf0, insem.at[0]).start()

    zero16 = jnp.zeros((16,), jnp.int32)

    def gbody(g, _unused):
        for rr in range(GROUP):
            m = g * GROUP + rr
            p = rr % 2
            np_ = (rr + 1) % 2
            row = row0 + m
            nxt = row0 + lax.rem(m + 1, rpw)
            pltpu.make_async_copy(keys_hbm.at[nxt], kbufs[np_],
                                  insem.at[np_]).start()
            pltpu.make_async_copy(keys_hbm.at[row], kbufs[p],
                                  insem.at[p]).wait()
            t = tbuf[pl.ds(m * 16, 16)]
            q = qbuf[pl.ds(m * 16, 16)]
            krow = kbufs[p]

            def vbody(j, carry):
                acc, eqb = carry
                v = krow[pl.ds(j * 16, 16)]
                m_gt = v > t
                m_eq = v == t
                c2 = plsc.cumsum(m_gt.astype(jnp.int32) * 65536
                                 + m_eq.astype(jnp.int32))
                cgt = c2 >> 16
                ceq = c2 & 0xFFFF
                rank = ceq + eqb
                m_sel = m_gt | (m_eq & (rank <= q))
                keep_incl = jnp.minimum(rank, q) - jnp.minimum(eqb, q)
                pos = (acc + cgt + keep_incl - m_sel.astype(jnp.int32)
                       + (rr * K))
                plsc.store_scatter(outk, [pos], v, mask=m_sel)
                plsc.store_scatter(outi, [pos], iota + j * 16, mask=m_sel)
                return (acc + plsc.all_reduce_population_count(m_sel),
                        eqb + plsc.all_reduce_population_count(m_eq))

            plsc.parallel_loop(0, POOL_SIZE // 16, 1, unroll=4,
                               carry=(zero16, zero16))(vbody)
        base = (row0 + g * GROUP) * K
        pltpu.sync_copy(outk, ck_hbm.at[pl.ds(base, GROUP * K)])
        pltpu.sync_copy(outi, ci_hbm.at[pl.ds(base, GROUP * K)])
        return _unused

    lax.fori_loop(0, rpw // GROUP, gbody, jnp.int32(0))
    pltpu.make_async_copy(keys_hbm.at[row0], kbuf0, insem.at[0]).wait()

  return _compact_body


def _compact_call(keys, t_b, quota_b):
    tokens = keys.shape[0]
    rpw = tokens // NW
    mesh = plsc.VectorSubcoreMesh(core_axis_name="c", subcore_axis_name="s", num_cores=2, num_subcores=16)
    f = pl.kernel(
        _make_compact_body(rpw),
        out_type=[
            jax.ShapeDtypeStruct((tokens * K,), jnp.int32),
            jax.ShapeDtypeStruct((tokens * K,), jnp.int32),
        ],
        mesh=mesh,
        compiler_params=pltpu.CompilerParams(needs_layout_passes=False),
        scratch_types=[
            pltpu.VMEM((POOL_SIZE,), jnp.int32),
            pltpu.VMEM((POOL_SIZE,), jnp.int32),
            pltpu.VMEM((rpw * 16,), jnp.int32),
            pltpu.VMEM((rpw * 16,), jnp.int32),
            pltpu.VMEM((GROUP * K,), jnp.int32),
            pltpu.VMEM((GROUP * K,), jnp.int32),
            pltpu.SemaphoreType.DMA((2,)),
        ],
    )
    ck, ci = f(keys, t_b.reshape(-1), quota_b.reshape(-1))
    return ck.reshape(tokens, K), ci.reshape(tokens, K)


def _bitonic_sort_desc(keys, idx):
    R, N = keys.shape
    lane = jax.lax.broadcasted_iota(jnp.int32, (R, N), 1)
    k = 2
    while k <= N:
        j = k // 2
        while j >= 1:
            low = (lane & j) == 0
            pk = jnp.where(low, pltpu.roll(keys, N - j, 1), pltpu.roll(keys, j, 1))
            pi = jnp.where(low, pltpu.roll(idx, N - j, 1), pltpu.roll(idx, j, 1))
            self_beats = (keys > pk) | ((keys == pk) & (idx < pi))
            want_max = ((lane & k) == 0) == low
            take_self = want_max == self_beats
            keys = jnp.where(take_self, keys, pk)
            idx = jnp.where(take_self, idx, pi)
            j //= 2
        k *= 2
    return keys, idx


def _sort_finalize_body(ckeys_ref, cidx_ref, comp_ref,
                        out_idx_ref, out_w_ref, out_mask_ref):
    keys, idx = _bitonic_sort_desc(ckeys_ref[...], cidx_ref[...])
    ts = _key_to_f32(keys)
    m = ts[:, 0:1]
    e = jnp.exp(ts - m)
    w = e / jnp.sum(e, axis=1, keepdims=True)
    comp = comp_ref[...]
    budgets = jnp.round(
        jnp.clip(MIN_P + (MAX_P - MIN_P) * comp * comp, MIN_P, MAX_P)
    ).astype(jnp.int32)
    pos = jax.lax.broadcasted_iota(jnp.int32, out_mask_ref.shape, 1)
    mask = (pos < budgets).astype(jnp.float32)
    out_idx_ref[...] = idx
    out_w_ref[...] = w * mask
    out_mask_ref[...] = mask


def _run_chunk(x, W1, b1, W2, b2, S1, bs1, S2, bs2):
    tokens = x.shape[0]
    n_ra = tokens // ROW_A
    hs, comp = pl.pallas_call(
        _hs_complexity_body,
        grid=(n_ra,),
        in_specs=[
            pl.BlockSpec((ROW_A, INPUT_DIM), lambda i: (i, 0)),
            pl.BlockSpec((INPUT_DIM, 128), lambda i: (0, 0)),
            pl.BlockSpec((128,), lambda i: (0,)),
            pl.BlockSpec((128, 1), lambda i: (0, 0)),
            pl.BlockSpec((1,), lambda i: (0,)),
            pl.BlockSpec((INPUT_DIM, HIDDEN_DIM), lambda i: (0, 0)),
            pl.BlockSpec((HIDDEN_DIM,), lambda i: (0,)),
        ],
        out_specs=[
            pl.BlockSpec((ROW_A, HIDDEN_DIM), lambda i: (i, 0)),
            pl.BlockSpec((ROW_A, 1), lambda i: (i, 0)),
        ],
        out_shape=[
            jax.ShapeDtypeStruct((tokens, HIDDEN_DIM), jnp.float32),
            jax.ShapeDtypeStruct((tokens, 1), jnp.float32),
        ],
    )(x, W1, b1, W2, b2, S1, bs1)

    n_rb = tokens // ROW_B
    keys, t_b, quota_b = pl.pallas_call(
        _keys_bisect_body,
        grid=(n_rb,),
        in_specs=[
            pl.BlockSpec((ROW_B, HIDDEN_DIM), lambda i: (i, 0)),
            pl.BlockSpec((HIDDEN_DIM, POOL_SIZE), lambda i: (0, 0)),
            pl.BlockSpec((POOL_SIZE,), lambda i: (0,)),
        ],
        out_specs=[
            pl.BlockSpec((ROW_B, POOL_SIZE), lambda i: (i, 0)),
            pl.BlockSpec((ROW_B, 16), lambda i: (i, 0)),
            pl.BlockSpec((ROW_B, 16), lambda i: (i, 0)),
        ],
        out_shape=[
            jax.ShapeDtypeStruct((tokens, POOL_SIZE), jnp.int32),
            jax.ShapeDtypeStruct((tokens, 16), jnp.int32),
            jax.ShapeDtypeStruct((tokens, 16), jnp.int32),
        ],
        scratch_shapes=[
            pltpu.VMEM((ROW_B, 1), jnp.int32),
            pltpu.VMEM((ROW_B, 1), jnp.int32),
            pltpu.VMEM((ROW_B, 1), jnp.int32),
        ],
    )(hs, S2, bs2)

    ckeys, cidx = _compact_call(keys, t_b, quota_b)

    n_rd = tokens // ROW_D
    out_idx, out_w, out_mask = pl.pallas_call(
        _sort_finalize_body,
        grid=(n_rd,),
        in_specs=[
            pl.BlockSpec((ROW_D, K), lambda i: (i, 0)),
            pl.BlockSpec((ROW_D, K), lambda i: (i, 0)),
            pl.BlockSpec((ROW_D, 1), lambda i: (i, 0)),
        ],
        out_specs=[
            pl.BlockSpec((ROW_D, K), lambda i: (i, 0)),
            pl.BlockSpec((ROW_D, K), lambda i: (i, 0)),
            pl.BlockSpec((ROW_D, K), lambda i: (i, 0)),
        ],
        out_shape=[
            jax.ShapeDtypeStruct((tokens, K), jnp.int32),
            jax.ShapeDtypeStruct((tokens, K), jnp.float32),
            jax.ShapeDtypeStruct((tokens, K), jnp.float32),
        ],
    )(ckeys, cidx, comp)
    return out_idx, out_w, out_mask, comp


CHUNKS = 4


def kernel(x, W1, b1, W2, b2, S1, bs1, S2, bs2):
    tc = TOKENS // CHUNKS
    parts = [
        _run_chunk(x[i * tc:(i + 1) * tc], W1, b1, W2, b2, S1, bs1, S2, bs2)
        for i in range(CHUNKS)
    ]
    return tuple(jnp.concatenate(leaves, axis=0) for leaves in zip(*parts))


# speedup vs baseline: 1.8080x; 1.0010x over previous
import jax
import jax.numpy as jnp
from jax import lax
from jax.experimental import pallas as pl
from jax.experimental.pallas import tpu as pltpu
from jax.experimental.pallas import tpu_sc as plsc

TOKENS = 8192
INPUT_DIM = 1024
HIDDEN_DIM = 256
POOL_SIZE = 16384
K = 1024
MIN_P = 100.0
MAX_P = 1024.0

ROW_A = 256
ROW_B = 128
ROW_D = 128
MAXI32 = 0x7FFFFFFF


def _hs_complexity_body(x_ref, W1_ref, b1_ref, W2_ref, b2_ref, S1_ref, bs1_ref,
                        hs_ref, comp_ref):
    x = x_ref[...]
    h = jnp.maximum(jnp.dot(x, W1_ref[...], preferred_element_type=jnp.float32)
                    + b1_ref[...], 0.0)
    logit = jnp.dot(h, W2_ref[...], preferred_element_type=jnp.float32) + b2_ref[...]
    comp_ref[...] = jax.nn.sigmoid(logit)
    hs_ref[...] = jnp.maximum(
        jnp.dot(x, S1_ref[...], preferred_element_type=jnp.float32) + bs1_ref[...], 0.0)


def _f32_to_key(s):
    b = jax.lax.bitcast_convert_type(s, jnp.int32)
    return jnp.where(b >= 0, b, MAXI32 - b)


def _key_to_f32(k):
    b = jnp.where(k >= 0, k, MAXI32 - k)
    return jax.lax.bitcast_convert_type(b, jnp.float32)


def _keys_bisect_body(hs_ref, S2_ref, bs2_ref, keys_ref, t_ref, quota_ref,
                      lo_ref, hi_ref, fr_ref):
    scores = (jnp.dot(hs_ref[...], S2_ref[...],
                      preferred_element_type=jnp.float32) + bs2_ref[...])
    keys = _f32_to_key(scores)
    keys_ref[...] = keys

    lo0 = jnp.min(keys, axis=1, keepdims=True)
    hi0 = jnp.max(keys, axis=1, keepdims=True) + 1

    mu = jnp.mean(scores, axis=1, keepdims=True)
    sd = jnp.sqrt(jnp.maximum(
        jnp.mean(scores * scores, axis=1, keepdims=True) - mu * mu, 0.0))
    c1 = _f32_to_key(mu + 1.1 * sd)
    c2 = _f32_to_key(mu + 2.0 * sd)
    n1 = jnp.sum((keys >= c1).astype(jnp.int32), axis=1, keepdims=True)
    n2 = jnp.sum((keys >= c2).astype(jnp.int32), axis=1, keepdims=True)
    lo = jnp.where(n1 >= K, c1, lo0)
    lo = jnp.where(n2 >= K, jnp.maximum(c2, lo), lo)
    hi = jnp.where(n2 < K, c2, hi0)
    hi = jnp.where(n1 < K, jnp.minimum(c1, hi), hi)
    fr0 = jnp.logical_or(n1 == K, n2 == K).astype(jnp.int32)
    lo = jnp.where(n1 == K, c1, lo)
    lo = jnp.where(n2 == K, c2, lo)
    lo_ref[...] = lo
    hi_ref[...] = hi
    fr_ref[...] = fr0

    def cond(carry):
        it, done = carry
        return jnp.logical_and(it < 34, done == 0)

    def body(carry):
        it, done = carry
        lo = lo_ref[...]
        hi = hi_ref[...]
        fr = fr_ref[...]
        mid = (lo >> 1) + (hi >> 1) + (lo & hi & 1)
        cnt = jnp.sum((keys >= mid).astype(jnp.int32), axis=1, keepdims=True)
        active = (fr == 0) & (hi - lo != 1)
        ge = cnt >= K
        new_lo = jnp.where(active & ge, mid, lo)
        new_hi = jnp.where(active & jnp.logical_not(ge), mid, hi)
        new_fr = fr | (active & (cnt == K)).astype(jnp.int32)
        lo_ref[...] = new_lo
        hi_ref[...] = new_hi
        fr_ref[...] = new_fr
        all_done = jnp.all((new_fr != 0) | (new_hi - new_lo == 1))
        return it + 1, all_done.astype(jnp.int32)

    jax.lax.while_loop(cond, body, (jnp.int32(0), jnp.int32(0)))

    lo = lo_ref[...]
    frozen = fr_ref[...] != 0
    sel = keys >= lo
    minsel = jnp.min(jnp.where(sel, keys, MAXI32), axis=1, keepdims=True)
    t = jnp.where(frozen, minsel, lo)
    cnt_gt = jnp.sum((keys > t).astype(jnp.int32), axis=1, keepdims=True)
    quota = K - cnt_gt
    t_ref[...] = jnp.broadcast_to(t, t_ref.shape)
    quota_ref[...] = jnp.broadcast_to(quota, quota_ref.shape)


NW = 32
GROUP = 8


def _make_compact_body(rpw):
  def _compact_body(keys_hbm, t_hbm, q_hbm, ck_hbm, ci_hbm,
                    kbuf0, kbuf1, tbuf, qbuf, outk, outi, insem):
    c = lax.axis_index("c")
    s = lax.axis_index("s")
    w = s * 2 + c
    row0 = w * rpw
    iota = lax.iota(jnp.int32, 16)
    pltpu.sync_copy(t_hbm.at[pl.ds(row0 * 16, rpw * 16)], tbuf)
    pltpu.sync_copy(q_hbm.at[pl.ds(row0 * 16, rpw * 16)], qbuf)
    kbufs = (kbuf0, kbuf1)
    pltpu.make_async_copy(keys_hbm.at[row0], kbuf0, insem.at[0]).start()

    zero16 = jnp.zeros((16,), jnp.int32)

    def gbody(g, _unused):
        for rr in range(GROUP):
            m = g * GROUP + rr
            p = rr % 2
            np_ = (rr + 1) % 2
            row = row0 + m
            nxt = row0 + lax.rem(m + 1, rpw)
            pltpu.make_async_copy(keys_hbm.at[nxt], kbufs[np_],
                                  insem.at[np_]).start()
            pltpu.make_async_copy(keys_hbm.at[row], kbufs[p],
                                  insem.at[p]).wait()
            t = tbuf[pl.ds(m * 16, 16)]
            q = qbuf[pl.ds(m * 16, 16)]
            krow = kbufs[p]

            def vbody(j, carry):
                acc, eqb = carry
                v = krow[pl.ds(j * 16, 16)]
                m_gt = v > t
                m_eq = v == t
                c2 = plsc.cumsum(m_gt.astype(jnp.int32) * 65536
                                 + m_eq.astype(jnp.int32))
                cgt = c2 >> 16
                ceq = c2 & 0xFFFF
                rank = ceq + eqb
                m_sel = m_gt | (m_eq & (rank <= q))
                keep_incl = jnp.minimum(rank, q) - jnp.minimum(eqb, q)
                pos = (acc + cgt + keep_incl - m_sel.astype(jnp.int32)
                       + (rr * K))
                plsc.store_scatter(outk, [pos], v, mask=m_sel)
                plsc.store_scatter(outi, [pos], iota + j * 16, mask=m_sel)
                return (acc + plsc.all_reduce_population_count(m_sel),
                        eqb + plsc.all_reduce_population_count(m_eq))

            plsc.parallel_loop(0, POOL_SIZE // 16, 1, unroll=4,
                               carry=(zero16, zero16))(vbody)
        base = (row0 + g * GROUP) * K
        pltpu.sync_copy(outk, ck_hbm.at[pl.ds(base, GROUP * K)])
        pltpu.sync_copy(outi, ci_hbm.at[pl.ds(base, GROUP * K)])
        return _unused

    lax.fori_loop(0, rpw // GROUP, gbody, jnp.int32(0))
    pltpu.make_async_copy(keys_hbm.at[row0], kbuf0, insem.at[0]).wait()

  return _compact_body


def _compact_call(keys, t_b, quota_b):
    tokens = keys.shape[0]
    rpw = tokens // NW
    mesh = plsc.VectorSubcoreMesh(core_axis_name="c", subcore_axis_name="s", num_cores=2, num_subcores=16)
    f = pl.kernel(
        _make_compact_body(rpw),
        out_type=[
            jax.ShapeDtypeStruct((tokens * K,), jnp.int32),
            jax.ShapeDtypeStruct((tokens * K,), jnp.int32),
        ],
        mesh=mesh,
        compiler_params=pltpu.CompilerParams(needs_layout_passes=False),
        scratch_types=[
            pltpu.VMEM((POOL_SIZE,), jnp.int32),
            pltpu.VMEM((POOL_SIZE,), jnp.int32),
            pltpu.VMEM((rpw * 16,), jnp.int32),
            pltpu.VMEM((rpw * 16,), jnp.int32),
            pltpu.VMEM((GROUP * K,), jnp.int32),
            pltpu.VMEM((GROUP * K,), jnp.int32),
            pltpu.SemaphoreType.DMA((2,)),
        ],
    )
    ck, ci = f(keys, t_b.reshape(-1), quota_b.reshape(-1))
    return ck.reshape(tokens, K), ci.reshape(tokens, K)


def _bitonic_sort_desc(keys, idx):
    R, N = keys.shape
    lane = jax.lax.broadcasted_iota(jnp.int32, (R, N), 1)
    k = 2
    while k <= N:
        j = k // 2
        while j >= 1:
            low = (lane & j) == 0
            pk = jnp.where(low, pltpu.roll(keys, N - j, 1), pltpu.roll(keys, j, 1))
            pi = jnp.where(low, pltpu.roll(idx, N - j, 1), pltpu.roll(idx, j, 1))
            self_beats = (keys > pk) | ((keys == pk) & (idx < pi))
            want_max = ((lane & k) == 0) == low
            take_self = want_max == self_beats
            keys = jnp.where(take_self, keys, pk)
            idx = jnp.where(take_self, idx, pi)
            j //= 2
        k *= 2
    return keys, idx


def _sort_finalize_body(ckeys_ref, cidx_ref, comp_ref,
                        out_idx_ref, out_w_ref, out_mask_ref):
    keys, idx = _bitonic_sort_desc(ckeys_ref[...], cidx_ref[...])
    ts = _key_to_f32(keys)
    m = ts[:, 0:1]
    e = jnp.exp(ts - m)
    w = e / jnp.sum(e, axis=1, keepdims=True)
    comp = comp_ref[...]
    budgets = jnp.round(
        jnp.clip(MIN_P + (MAX_P - MIN_P) * comp * comp, MIN_P, MAX_P)
    ).astype(jnp.int32)
    pos = jax.lax.broadcasted_iota(jnp.int32, out_mask_ref.shape, 1)
    mask = (pos < budgets).astype(jnp.float32)
    out_idx_ref[...] = idx
    out_w_ref[...] = w * mask
    out_mask_ref[...] = mask


def _run_chunk(x, W1, b1, W2, b2, S1, bs1, S2, bs2):
    tokens = x.shape[0]
    n_ra = tokens // ROW_A
    hs, comp = pl.pallas_call(
        _hs_complexity_body,
        grid=(n_ra,),
        in_specs=[
            pl.BlockSpec((ROW_A, INPUT_DIM), lambda i: (i, 0)),
            pl.BlockSpec((INPUT_DIM, 128), lambda i: (0, 0)),
            pl.BlockSpec((128,), lambda i: (0,)),
            pl.BlockSpec((128, 1), lambda i: (0, 0)),
            pl.BlockSpec((1,), lambda i: (0,)),
            pl.BlockSpec((INPUT_DIM, HIDDEN_DIM), lambda i: (0, 0)),
            pl.BlockSpec((HIDDEN_DIM,), lambda i: (0,)),
        ],
        out_specs=[
            pl.BlockSpec((ROW_A, HIDDEN_DIM), lambda i: (i, 0)),
            pl.BlockSpec((ROW_A, 1), lambda i: (i, 0)),
        ],
        out_shape=[
            jax.ShapeDtypeStruct((tokens, HIDDEN_DIM), jnp.float32),
            jax.ShapeDtypeStruct((tokens, 1), jnp.float32),
        ],
    )(x, W1, b1, W2, b2, S1, bs1)

    n_rb = tokens // ROW_B
    keys, t_b, quota_b = pl.pallas_call(
        _keys_bisect_body,
        grid=(n_rb,),
        in_specs=[
            pl.BlockSpec((ROW_B, HIDDEN_DIM), lambda i: (i, 0)),
            pl.BlockSpec((HIDDEN_DIM, POOL_SIZE), lambda i: (0, 0)),
            pl.BlockSpec((POOL_SIZE,), lambda i: (0,)),
        ],
        out_specs=[
            pl.BlockSpec((ROW_B, POOL_SIZE), lambda i: (i, 0)),
            pl.BlockSpec((ROW_B, 16), lambda i: (i, 0)),
            pl.BlockSpec((ROW_B, 16), lambda i: (i, 0)),
        ],
        out_shape=[
            jax.ShapeDtypeStruct((tokens, POOL_SIZE), jnp.int32),
            jax.ShapeDtypeStruct((tokens, 16), jnp.int32),
            jax.ShapeDtypeStruct((tokens, 16), jnp.int32),
        ],
        scratch_shapes=[
            pltpu.VMEM((ROW_B, 1), jnp.int32),
            pltpu.VMEM((ROW_B, 1), jnp.int32),
            pltpu.VMEM((ROW_B, 1), jnp.int32),
        ],
    )(hs, S2, bs2)

    ckeys, cidx = _compact_call(keys, t_b, quota_b)

    n_rd = tokens // ROW_D
    out_idx, out_w, out_mask = pl.pallas_call(
        _sort_finalize_body,
        grid=(n_rd,),
        in_specs=[
            pl.BlockSpec((ROW_D, K), lambda i: (i, 0)),
            pl.BlockSpec((ROW_D, K), lambda i: (i, 0)),
            pl.BlockSpec((ROW_D, 1), lambda i: (i, 0)),
        ],
        out_specs=[
            pl.BlockSpec((ROW_D, K), lambda i: (i, 0)),
            pl.BlockSpec((ROW_D, K), lambda i: (i, 0)),
            pl.BlockSpec((ROW_D, K), lambda i: (i, 0)),
        ],
        out_shape=[
            jax.ShapeDtypeStruct((tokens, K), jnp.int32),
            jax.ShapeDtypeStruct((tokens, K), jnp.float32),
            jax.ShapeDtypeStruct((tokens, K), jnp.float32),
        ],
    )(ckeys, cidx, comp)
    return out_idx, out_w, out_mask, comp


CHUNKS = 4


def kernel(x, W1, b1, W2, b2, S1, bs1, S2, bs2):
    tc = TOKENS // CHUNKS
    parts = [
        _run_chunk(x[i * tc:(i + 1) * tc], W1, b1, W2, b2, S1, bs1, S2, bs2)
        for i in range(CHUNKS)
    ]
    return tuple(jnp.concatenate(leaves, axis=0) for leaves in zip(*parts))
